# baseline (speedup 1.0000x reference)
import functools

import jax
import jax.numpy as jnp
from jax import lax
from jax.experimental import pallas as pl
from jax.experimental.pallas import tpu as pltpu
from jax.experimental.pallas import tpu_sc as plsc

_E = 64
_K = 8
_NC = 2
_NS = 16
_NW = _NC * _NS
_L = 16


def _probs_block(x_ref, w_ref, probs_ref, probs_t_ref):
    x = x_ref[...]
    w = w_ref[...]
    lgT = lax.dot_general(
        w, x, (((1,), (1,)), ((), ())),
        preferred_element_type=jnp.float32)
    m = jnp.max(lgT, axis=0, keepdims=True)
    e = jnp.exp(lgT - m)
    s = jnp.sum(e, axis=0, keepdims=True)
    p = e * (1.0 / s)
    probs_ref[...] = p.T
    probs_t_ref[...] = p


def _tc_probs(N, D):
    TB = 4096
    return pl.pallas_call(
        _probs_block,
        grid=(N // TB,),
        in_specs=[
            pl.BlockSpec((TB, D), lambda i: (i, 0)),
            pl.BlockSpec((_E, D), lambda i: (0, 0)),
        ],
        out_specs=[
            pl.BlockSpec((TB, _E), lambda i: (i, 0)),
            pl.BlockSpec((_E, TB), lambda i: (0, i)),
        ],
        out_shape=[
            jax.ShapeDtypeStruct((N, _E), jnp.float32),
            jax.ShapeDtypeStruct((_E, N), jnp.float32),
        ],
        compiler_params=pltpu.CompilerParams(
            dimension_semantics=("arbitrary",),
        ),
    )


def _sc_topk(n_tokens):
    tpw = n_tokens // _NW
    ngroups = tpw // _L
    mesh = plsc.VectorSubcoreMesh(core_axis_name="c", subcore_axis_name="s")

    @functools.partial(
        pl.kernel,
        out_type=[
            jax.ShapeDtypeStruct((_K * n_tokens,), jnp.float32),
            jax.ShapeDtypeStruct((_K * n_tokens,), jnp.int32),
        ],
        mesh=mesh,
        scratch_types=[
            pltpu.VMEM((_E * tpw,), jnp.float32),
            pltpu.VMEM((_K * tpw,), jnp.float32),
            pltpu.VMEM((_K * tpw,), jnp.int32),
            pltpu.SemaphoreType.DMA,
        ],
    )
    def sc_kernel(apt_hbm, tp_hbm, ti_hbm, pv, tp_v, ti_v, sem):
        wid = lax.axis_index("s") * _NC + lax.axis_index("c")
        base = wid * tpw
        copies = [
            pltpu.async_copy(apt_hbm.at[pl.ds(e * n_tokens + base, tpw)],
                             pv.at[pl.ds(e * tpw, tpw)], sem)
            for e in range(_E)
        ]
        for c in copies:
            c.wait()

        def group_body(g, carry):
            t0 = g * _L
            denom = jnp.full((_L,), 1e-9, jnp.float32)
            vs = [pv[pl.ds(e * tpw + t0, _L)] for e in range(_E)]
            tvals = []
            tidxs = []
            for _ in range(_K):
                lv = list(vs)
                li = [jnp.full((_L,), e, jnp.int32) for e in range(_E)]
                while len(lv) > 1:
                    nv, ni = [], []
                    for i in range(0, len(lv), 2):
                        gt = lv[i + 1] > lv[i]
                        nv.append(jnp.where(gt, lv[i + 1], lv[i]))
                        ni.append(jnp.where(gt, li[i + 1], li[i]))
                    lv, li = nv, ni
                mval, midx = lv[0], li[0]
                vs = [jnp.where(midx == e, -1.0, vs[e]) for e in range(_E)]
                tvals.append(mval)
                tidxs.append(midx)
                denom = denom + mval
            rdenom = 1.0 / denom
            for k in range(_K):
                tp_v[pl.ds(k * tpw + t0, _L)] = tvals[k] * rdenom
                ti_v[pl.ds(k * tpw + t0, _L)] = tidxs[k]
            return carry

        lax.fori_loop(0, ngroups, group_body, 0)

        out_copies = []
        for k in range(_K):
            out_copies.append(
                pltpu.async_copy(tp_v.at[pl.ds(k * tpw, tpw)],
                                 tp_hbm.at[pl.ds(k * n_tokens + base, tpw)],
                                 sem))
            out_copies.append(
                pltpu.async_copy(ti_v.at[pl.ds(k * tpw, tpw)],
                                 ti_hbm.at[pl.ds(k * n_tokens + base, tpw)],
                                 sem))
        for c in out_copies:
            c.wait()

    return sc_kernel


def kernel(x, W):
    B, S, D = x.shape
    N = B * S
    xf = x.reshape(N, D)
    ap, apt = _tc_probs(N, D)(xf, W)
    tp_f, ti_f = _sc_topk(N)(apt.reshape(-1))
    tp = tp_f.reshape(_K, N).T
    ti = ti_f.reshape(_K, N).T
    return (tp.reshape(B, S, _K), ti.reshape(B, S, _K), ap.reshape(B, S, _E))


# speedup vs baseline: 1.7452x; 1.7452x over previous
import jax
import jax.numpy as jnp
from jax.experimental import pallas as pl
from jax.experimental.pallas import tpu as pltpu

_E = 64
_K = 8


def _router_block(x_ref, w_ref, topk_p_ref, topk_i_ref, probs_ref):
    x = x_ref[...]
    w = w_ref[...]
    lgT = jax.lax.dot_general(
        w, x, (((1,), (1,)), ((), ())),
        preferred_element_type=jnp.float32)

    iota_s = jax.lax.broadcasted_iota(
        jnp.int32, lgT.shape, 0).astype(jnp.float32)
    m = jnp.max(lgT, axis=0, keepdims=True)
    e = jnp.exp(lgT - m)
    s = jnp.sum(e, axis=0, keepdims=True)
    p = e * (1.0 / s)
    probs_ref[...] = p.T

    work = p
    vals = []
    idxs = []
    for _ in range(_K):
        mx = jnp.max(work, axis=0, keepdims=True)
        ixf = jnp.min(jnp.where(work == mx, iota_s, float(_E)),
                      axis=0, keepdims=True)
        work = jnp.where(iota_s == ixf, -1.0, work)
        vals.append(mx)
        idxs.append(ixf)
    tv = jnp.concatenate(vals, axis=0)
    ti = jnp.concatenate(idxs, axis=0)
    denom = jnp.sum(tv, axis=0, keepdims=True) + 1e-9
    topk_p_ref[...] = (tv / denom).T
    topk_i_ref[...] = ti.astype(jnp.int32).T


def kernel(x, W):
    B, S, D = x.shape
    N = B * S
    xf = x.reshape(N, D)

    TB = 4096
    grid = (N // TB,)
    tp, ti, ap = pl.pallas_call(
        _router_block,
        grid=grid,
        in_specs=[
            pl.BlockSpec((TB, D), lambda i: (i, 0)),
            pl.BlockSpec((_E, D), lambda i: (0, 0)),
        ],
        out_specs=[
            pl.BlockSpec((TB, _K), lambda i: (i, 0)),
            pl.BlockSpec((TB, _K), lambda i: (i, 0)),
            pl.BlockSpec((TB, _E), lambda i: (i, 0)),
        ],
        out_shape=[
            jax.ShapeDtypeStruct((N, _K), jnp.float32),
            jax.ShapeDtypeStruct((N, _K), jnp.int32),
            jax.ShapeDtypeStruct((N, _E), jnp.float32),
        ],
        compiler_params=pltpu.CompilerParams(
            dimension_semantics=("arbitrary",),
        ),
    )(xf, W)
    return (tp.reshape(B, S, _K), ti.reshape(B, S, _K), ap.reshape(B, S, _E))
